# baseline (speedup 1.0000x reference)
import functools

import jax
import jax.numpy as jnp
from jax import lax
from jax.experimental import pallas as pl
from jax.experimental.pallas import tpu as pltpu
from jax.experimental.pallas import tpu_sc as plsc

R, C = 128, 32768
L = 16
NC, NS = 2, 16
NW = NC * NS
ROWS_PER_W = R // NW
U = 8
STRIDE = U * L
NSPLIT = 4
CSP = C // NSPLIT


def _scan_row(inbuf, lanes):
    neg_inf = jnp.full((L,), -jnp.inf, jnp.float32)
    zero_i = jnp.zeros((L,), jnp.int32)

    def body(i, carry):
        vmaxs, vidxs, cidx = carry
        new_vmaxs = []
        new_vidxs = []
        for a in range(U):
            v = inbuf[pl.ds(i * STRIDE + a * L, L)]
            m = v > vmaxs[a]
            new_vmaxs.append(jnp.where(m, v, vmaxs[a]))
            new_vidxs.append(jnp.where(m, cidx, vidxs[a]))
        return tuple(new_vmaxs), tuple(new_vidxs), cidx + STRIDE

    vmaxs, vidxs, _ = lax.fori_loop(
        0,
        C // STRIDE,
        body,
        ((neg_inf,) * U, (zero_i,) * U, zero_i),
    )
    vidxs = [vidxs[a] + (a * L) + lanes for a in range(U)]
    vmaxs = list(vmaxs)
    n = U
    while n > 1:
        n //= 2
        for a in range(n):
            ov, oi = vmaxs[a + n], vidxs[a + n]
            better = (ov > vmaxs[a]) | ((ov == vmaxs[a]) & (oi < vidxs[a]))
            vmaxs[a] = jnp.where(better, ov, vmaxs[a])
            vidxs[a] = jnp.where(better, oi, vidxs[a])
    return vmaxs[0], vidxs[0]


def _body(x_hbm, out_hbm, inbuf0, inbuf1, obuf, sv, si, sem_in, sem_out):
    wid = lax.axis_index("s") * NC + lax.axis_index("c")
    lanes = lax.iota(jnp.int32, L)
    zeros_v = jnp.zeros((L,), jnp.float32)
    ones_v = jnp.ones((L,), jnp.float32)
    lane0 = lanes == 0
    r0 = wid * ROWS_PER_W
    inbufs = [inbuf0, inbuf1]

    def start_in(row, buf):
        return [
            pltpu.async_copy(
                x_hbm.at[row, pl.ds(j * CSP, CSP)],
                buf.at[pl.ds(j * CSP, CSP)],
                sem_in,
            )
            for j in range(NSPLIT)
        ]

    def start_out(row):
        return [
            pltpu.async_copy(
                obuf.at[pl.ds(j * CSP, CSP)],
                out_hbm.at[row, pl.ds(j * CSP, CSP)],
                sem_out,
            )
            for j in range(NSPLIT)
        ]

    cp_in = start_in(r0, inbufs[0])

    def zbody(i, _):
        for a in range(U):
            obuf[pl.ds(i * STRIDE + a * L, L)] = zeros_v
        return 0

    lax.fori_loop(0, C // STRIDE, zbody, 0)

    out_cp = None
    prev_vidx = None
    for k in range(ROWS_PER_W):
        for cp in cp_in:
            cp.wait()
        if k + 1 < ROWS_PER_W:
            cp_in = start_in(r0 + k + 1, inbufs[(k + 1) % 2])
        vmax, vidx = _scan_row(inbufs[k % 2], lanes)

        for shift in (1, 2, 4, 8):
            sv[...] = vmax
            si[...] = vidx
            perm = lanes ^ shift
            ov = plsc.load_gather(sv, [perm])
            oi = plsc.load_gather(si, [perm])
            better = (ov > vmax) | ((ov == vmax) & (oi < vidx))
            vmax = jnp.where(better, ov, vmax)
            vidx = jnp.where(better, oi, vidx)

        if out_cp is not None:
            for cp in out_cp:
                cp.wait()
            plsc.store_scatter(obuf, [prev_vidx], zeros_v, mask=lane0)
        plsc.store_scatter(obuf, [vidx], ones_v, mask=lane0)
        out_cp = start_out(r0 + k)
        prev_vidx = vidx

    for cp in out_cp:
        cp.wait()


@jax.jit
def kernel(x):
    mesh = plsc.VectorSubcoreMesh(
        core_axis_name="c", subcore_axis_name="s", num_cores=NC, num_subcores=NS
    )
    f = functools.partial(
        pl.kernel,
        mesh=mesh,
        out_type=jax.ShapeDtypeStruct((R, C), jnp.float32),
        scratch_types=[
            pltpu.VMEM((C,), jnp.float32),
            pltpu.VMEM((C,), jnp.float32),
            pltpu.VMEM((C,), jnp.float32),
            pltpu.VMEM((L,), jnp.float32),
            pltpu.VMEM((L,), jnp.int32),
            pltpu.SemaphoreType.DMA,
            pltpu.SemaphoreType.DMA,
        ],
        compiler_params=pltpu.CompilerParams(needs_layout_passes=False),
    )(_body)
    return f(x)


# speedup vs baseline: 1.0056x; 1.0056x over previous
import functools

import jax
import jax.numpy as jnp
from jax import lax
from jax.experimental import pallas as pl
from jax.experimental.pallas import tpu as pltpu
from jax.experimental.pallas import tpu_sc as plsc

R, C = 128, 32768
L = 16
NC, NS = 2, 16
NW = NC * NS
ROWS_PER_W = R // NW
U = 8
STRIDE = U * L


def _scan_row(inbuf, lanes):
    neg_inf = jnp.full((L,), -jnp.inf, jnp.float32)
    zero_i = jnp.zeros((L,), jnp.int32)

    def body(i, carry):
        vmaxs, vidxs, cidx = carry
        new_vmaxs = []
        new_vidxs = []
        for a in range(U):
            v = inbuf[pl.ds(i * STRIDE + a * L, L)]
            m = v > vmaxs[a]
            new_vmaxs.append(jnp.where(m, v, vmaxs[a]))
            new_vidxs.append(jnp.where(m, cidx, vidxs[a]))
        return tuple(new_vmaxs), tuple(new_vidxs), cidx + STRIDE

    vmaxs, vidxs, _ = lax.fori_loop(
        0,
        C // STRIDE,
        body,
        ((neg_inf,) * U, (zero_i,) * U, zero_i),
    )
    vidxs = [vidxs[a] + (a * L) + lanes for a in range(U)]
    vmaxs = list(vmaxs)
    n = U
    while n > 1:
        n //= 2
        for a in range(n):
            ov, oi = vmaxs[a + n], vidxs[a + n]
            better = (ov > vmaxs[a]) | ((ov == vmaxs[a]) & (oi < vidxs[a]))
            vmaxs[a] = jnp.where(better, ov, vmaxs[a])
            vidxs[a] = jnp.where(better, oi, vidxs[a])
    gmax = jnp.max(vmaxs[0])
    cand = jnp.where(vmaxs[0] == gmax, vidxs[0], jnp.int32(C))
    return jnp.min(cand)


def _body(x_hbm, out_hbm, inbuf0, inbuf1, zbuf, pbuf, sem_in, sem_z, sem_p):
    wid = lax.axis_index("s") * NC + lax.axis_index("c")
    lanes = lax.iota(jnp.int32, L)
    zeros_v = jnp.zeros((L,), jnp.float32)
    ones_v = jnp.ones((L,), jnp.float32)
    lane0 = lanes == 0
    r0 = wid * ROWS_PER_W
    inbufs = [inbuf0, inbuf1]

    cp_in = pltpu.async_copy(x_hbm.at[r0], inbufs[0], sem_in)

    def zbody(i, _):
        for a in range(U):
            zbuf[pl.ds(i * STRIDE + a * L, L)] = zeros_v
        return 0

    lax.fori_loop(0, C // STRIDE, zbody, 0)
    for k in range(ROWS_PER_W):
        pbuf[pl.ds(k * L, L)] = zeros_v

    zcps = [
        pltpu.async_copy(zbuf, out_hbm.at[r0 + k], sem_z)
        for k in range(ROWS_PER_W)
    ]

    segs = []
    for k in range(ROWS_PER_W):
        cp_in.wait()
        if k + 1 < ROWS_PER_W:
            cp_in = pltpu.async_copy(
                x_hbm.at[r0 + k + 1], inbufs[(k + 1) % 2], sem_in
            )
        idx = _scan_row(inbufs[k % 2], lanes)
        seg = pl.multiple_of((idx // L) * L, L)
        off = jnp.full((L,), k * L, jnp.int32) + (idx - seg)
        plsc.store_scatter(pbuf, [off], ones_v, mask=lane0)
        segs.append(seg)

    for z in zcps:
        z.wait()
    pcps = [
        pltpu.async_copy(
            pbuf.at[pl.ds(k * L, L)],
            out_hbm.at[r0 + k, pl.ds(segs[k], L)],
            sem_p,
        )
        for k in range(ROWS_PER_W)
    ]
    for p in pcps:
        p.wait()


@jax.jit
def kernel(x):
    mesh = plsc.VectorSubcoreMesh(
        core_axis_name="c", subcore_axis_name="s", num_cores=NC, num_subcores=NS
    )
    f = functools.partial(
        pl.kernel,
        mesh=mesh,
        out_type=jax.ShapeDtypeStruct((R, C), jnp.float32),
        scratch_types=[
            pltpu.VMEM((C,), jnp.float32),
            pltpu.VMEM((C,), jnp.float32),
            pltpu.VMEM((C,), jnp.float32),
            pltpu.VMEM((ROWS_PER_W * L,), jnp.float32),
            pltpu.SemaphoreType.DMA,
            pltpu.SemaphoreType.DMA,
            pltpu.SemaphoreType.DMA,
        ],
        compiler_params=pltpu.CompilerParams(needs_layout_passes=False),
    )(_body)
    return f(x)
